# baseline (speedup 1.0000x reference)
import functools

import jax
import jax.numpy as jnp
from jax import lax
from jax.experimental import pallas as pl
from jax.experimental.pallas import tpu as pltpu
from jax.experimental.pallas import tpu_sc as plsc

N_USERS = 25000
N_NODES = 50000
H = 32
HP = 32

NS = 16
NC = 2
NACC = 50008
RT = N_NODES // NS
WB = 125
NWB = RT // WB
CH = 128
NPC = 8
SUP = CH * NPC
EPT = 51200
NSUP = EPT // SUP
EP = EPT * NS
N_EDGES = 800000

_mesh = plsc.VectorSubcoreMesh(core_axis_name="c", subcore_axis_name="s")


def _decode(u):
    return plsc.unpack(u, format=plsc.PackFormat.INTERLEAVED)


def _encode(a, b):
    return plsc.pack(a, b, format=plsc.PackFormat.INTERLEAVED)


@functools.partial(
    pl.kernel,
    out_type=[
        jax.ShapeDtypeStruct((NC * N_NODES, H), jnp.float32),
        jax.ShapeDtypeStruct((NC * N_NODES, HP), jnp.bfloat16),
    ],
    mesh=_mesh,
    scratch_types=[
        pltpu.VMEM_SHARED((NACC, H), jnp.float32),
        pltpu.VMEM((NPC, 128), jnp.int32),
        pltpu.VMEM((NPC, 128), jnp.int32),
        pltpu.VMEM((SUP,), jnp.float32),
        pltpu.VMEM((2, CH, HP), jnp.bfloat16),
        pltpu.VMEM((2, CH, H), jnp.float32),
        pltpu.VMEM((WB, H), jnp.float32),
        pltpu.SemaphoreType.DMA,
        pltpu.SemaphoreType.DMA,
        pltpu.SemaphoreType.DMA,
        pltpu.SemaphoreType.DMA,
    ],
    compiler_params=pltpu.CompilerParams(use_tc_tiling_on_sc=False, needs_layout_passes=False),
)
def _lightgcn_sc(emb_pk, emb_f32, col2d, row2d, w_hbm, out, ebuf, acc,
                 colv, rowv, wv, gbuf, sbuf, abuf, sem0, sem1, sem_i, sem_s):
    c = lax.axis_index("c")
    s = lax.axis_index("s")
    coff = c * N_NODES
    row_base = s * RT
    sems = (sem0, sem1)

    def layer(src_tab, old_tab, is_last):
        def zbody(r, _):
            z = jnp.zeros((16,), jnp.float32)
            abuf[r, 0:16] = z
            abuf[r, 16:32] = z
            return 0
        lax.fori_loop(0, WB, zbody, 0)
        for m in range(NWB):
            pltpu.sync_copy(abuf, acc.at[pl.ds(row_base + m * WB, WB)])

        @pl.when(s == 0)
        def _():
            pltpu.sync_copy(abuf.at[pl.ds(0, 8)], acc.at[pl.ds(N_NODES, 8)])
        plsc.subcore_barrier()

        def sup_body(t, _):
            erow = s * (EPT // 128) + t * NPC
            ebase = s * EPT + t * SUP
            idx_descs = [
                pltpu.async_copy(col2d.at[pl.ds(erow, NPC)], colv, sem_i),
                pltpu.async_copy(row2d.at[pl.ds(erow, NPC)], rowv, sem_i),
                pltpu.async_copy(w_hbm.at[pl.ds(ebase, SUP)], wv, sem_i),
            ]
            for d in idx_descs:
                d.wait()
            def cadd(j, _):
                for i in range(8):
                    colv[j, 16 * i:16 * (i + 1)] = (
                        colv[j, 16 * i:16 * (i + 1)] + coff)
                return 0
            lax.fori_loop(0, NPC, cadd, 0)

            def fire(cc):
                slot = cc % 2
                return pltpu.async_copy(src_tab.at[colv.at[cc]],
                                        gbuf.at[slot], sems[slot])

            desc = fire(0)
            sc_prev = None
            for cc in range(NPC):
                slot = cc % 2
                if sc_prev is not None:
                    sc_prev.wait()
                nxt = fire(cc + 1) if cc + 1 < NPC else None
                desc.wait()
                desc = nxt

                def scale(g, _):
                    wvec = wv[pl.ds(cc * CH + g * 16, 16)]
                    for u in range(16):
                        r = g * 16 + u
                        ws = wvec.at[jnp.full((16,), u, jnp.int32)].get(
                            mode="promise_in_bounds")
                        a, b = _decode(gbuf[slot, r, 0:32])
                        sbuf[slot, r, 0:16] = a * ws
                        sbuf[slot, r, 16:32] = b * ws
                    return 0
                lax.fori_loop(0, CH // 16, scale, 0)

                sc_prev = pltpu.async_copy(sbuf.at[slot],
                                           acc.at[rowv.at[cc]], sem_s,
                                           add=True)
            sc_prev.wait()
            return 0

        lax.fori_loop(0, NSUP, sup_body, 0)
        plsc.subcore_barrier()

        for m in range(NWB):
            off = row_base + m * WB
            hoff = coff + off
            pltpu.sync_copy(acc.at[pl.ds(off, WB)],
                            sbuf.at[0].at[pl.ds(0, WB)])
            pltpu.sync_copy(old_tab.at[pl.ds(hoff, WB)], abuf)

            def accum(r, _):
                n0 = sbuf[0, r, 0:16]
                n1 = sbuf[0, r, 16:32]
                a0 = abuf[r, 0:16] + n0
                a1 = abuf[r, 16:32] + n1
                if is_last:
                    third = jnp.float32(1.0 / 3.0)
                    a0 = a0 * third
                    a1 = a1 * third
                else:
                    gbuf[0, r, 0:32] = _encode(n0, n1)
                abuf[r, 0:16] = a0
                abuf[r, 16:32] = a1
                return 0
            lax.fori_loop(0, WB, accum, 0)

            pltpu.sync_copy(abuf, out.at[pl.ds(hoff, WB)])
            if not is_last:
                pltpu.sync_copy(gbuf.at[0].at[pl.ds(0, WB)],
                                ebuf.at[pl.ds(hoff, WB)])
        plsc.subcore_barrier()

    layer(emb_pk, emb_f32, is_last=False)
    layer(ebuf, out, is_last=True)


def _pack_host(v):
    inter = jnp.stack([v[:, 0:16], v[:, 16:32]], axis=2).reshape(v.shape[0], 32)
    return inter.astype(jnp.bfloat16)


def kernel(embedding, edge_weight, edge_index):
    row = edge_index[0].astype(jnp.int32)
    col = edge_index[1].astype(jnp.int32)
    w = edge_weight.astype(jnp.float32)

    emb2 = jnp.stack([embedding[:, :H], embedding[:, H:]], axis=0)
    emb2 = emb2.reshape(NC * N_NODES, H)
    emb_pk = _pack_host(emb2)

    colp = jnp.zeros((EP,), jnp.int32).at[:N_EDGES].set(col).reshape(EP // 128, 128)
    rowp = jnp.full((EP,), N_NODES, jnp.int32).at[:N_EDGES].set(row).reshape(EP // 128, 128)
    wp = jnp.zeros((EP,), jnp.float32).at[:N_EDGES].set(w)

    out, _ = _lightgcn_sc(emb_pk, emb2, colp, rowp, wp)
    halves = out.reshape(NC, N_NODES, H)
    e_final = jnp.concatenate([halves[0], halves[1]], axis=1)
    return (e_final[:N_USERS], e_final[N_USERS:])


# speedup vs baseline: 1.3211x; 1.3211x over previous
import functools

import jax
import jax.numpy as jnp
from jax import lax
from jax.experimental import pallas as pl
from jax.experimental.pallas import tpu as pltpu
from jax.experimental.pallas import tpu_sc as plsc

N_USERS = 25000
N_NODES = 50000
H = 32

NS = 16
NC = 2
NACC = 50008
RT = N_NODES // NS
WB = 125
NWB = RT // WB
CH = 256
NPC = 8
SUP = CH * NPC
EPT = 51200
NSUP = EPT // SUP
EP = EPT * NS
N_EDGES = 800000

_mesh = plsc.VectorSubcoreMesh(core_axis_name="c", subcore_axis_name="s")


@functools.partial(
    pl.kernel,
    out_type=[
        jax.ShapeDtypeStruct((NC * N_NODES, H), jnp.float32),
        jax.ShapeDtypeStruct((NC * N_NODES, H), jnp.float32),
    ],
    mesh=_mesh,
    scratch_types=[
        pltpu.VMEM_SHARED((NACC, H), jnp.float32),
        pltpu.VMEM((SUP,), jnp.int32),
        pltpu.VMEM((NPC, CH), jnp.int32),
        pltpu.VMEM((SUP,), jnp.float32),
        pltpu.VMEM((2, CH, H), jnp.float32),
        pltpu.VMEM((WB, H), jnp.float32),
        pltpu.SemaphoreType.DMA,
        pltpu.SemaphoreType.DMA,
        pltpu.SemaphoreType.DMA,
        pltpu.SemaphoreType.DMA,
    ],
    compiler_params=pltpu.CompilerParams(use_tc_tiling_on_sc=False),
)
def _lightgcn_sc(emb2, col_hbm, row2d, w_hbm, out, ebuf, acc, colv, rowv, wv,
                 gbuf, abuf, sem0, sem1, sem_i, sem_s):
    c = lax.axis_index("c")
    s = lax.axis_index("s")
    coff = c * N_NODES
    row_base = s * RT
    sems = (sem0, sem1)

    def layer(src_tab, old_tab, is_last):
        def zbody(r, _):
            z = jnp.zeros((16,), jnp.float32)
            abuf[r, 0:16] = z
            abuf[r, 16:32] = z
            return 0
        lax.fori_loop(0, WB, zbody, 0)
        for m in range(NWB):
            pltpu.sync_copy(abuf, acc.at[pl.ds(row_base + m * WB, WB)])

        @pl.when(s == 0)
        def _():
            pltpu.sync_copy(abuf.at[pl.ds(0, 8)], acc.at[pl.ds(N_NODES, 8)])
        plsc.subcore_barrier()

        def sup_body(t, _):
            erow = s * (EPT // CH) + t * NPC
            ebase = s * EPT + t * SUP
            idx_descs = [
                pltpu.async_copy(col_hbm.at[pl.ds(ebase, SUP)], colv, sem_i),
                pltpu.async_copy(row2d.at[pl.ds(erow, NPC)], rowv, sem_i),
                pltpu.async_copy(w_hbm.at[pl.ds(ebase, SUP)], wv, sem_i),
            ]
            for d in idx_descs:
                d.wait()
            def cadd(j, _):
                for i in range(4):
                    base = j * 64 + i * 16
                    colv[pl.ds(base, 16)] = colv[pl.ds(base, 16)] + coff
                return 0
            lax.fori_loop(0, SUP // 64, cadd, 0)

            def fire(cc):
                slot = cc % 2
                return [
                    pltpu.async_copy(
                        src_tab.at[colv.at[pl.ds(cc * CH, CH)]],
                        gbuf.at[slot], sems[slot])
                ]

            descs = fire(0)
            sc_prev = None
            for cc in range(NPC):
                slot = cc % 2
                if sc_prev is not None:
                    for d in sc_prev:
                        d.wait()
                nxt = fire(cc + 1) if cc + 1 < NPC else None
                for d in descs:
                    d.wait()
                descs = nxt

                def scale(g, _):
                    wvec = wv[pl.ds(cc * CH + g * 16, 16)]
                    for u in range(16):
                        r = g * 16 + u
                        ws = wvec.at[jnp.full((16,), u, jnp.int32)].get(
                            mode="promise_in_bounds")
                        gbuf[slot, r, 0:16] = gbuf[slot, r, 0:16] * ws
                        gbuf[slot, r, 16:32] = gbuf[slot, r, 16:32] * ws
                    return 0
                lax.fori_loop(0, CH // 16, scale, 0)

                sc_prev = [
                    pltpu.async_copy(gbuf.at[slot],
                                     acc.at[rowv.at[cc]], sem_s,
                                     add=True)
                ]
            for d in sc_prev:
                d.wait()
            return 0

        lax.fori_loop(0, NSUP, sup_body, 0)
        plsc.subcore_barrier()

        for m in range(NWB):
            off = row_base + m * WB
            hoff = coff + off
            pltpu.sync_copy(acc.at[pl.ds(off, WB)],
                            gbuf.at[0].at[pl.ds(0, WB)])
            pltpu.sync_copy(old_tab.at[pl.ds(hoff, WB)], abuf)

            def accum(r, _):
                a0 = abuf[r, 0:16] + gbuf[0, r, 0:16]
                a1 = abuf[r, 16:32] + gbuf[0, r, 16:32]
                if is_last:
                    third = jnp.float32(1.0 / 3.0)
                    a0 = a0 * third
                    a1 = a1 * third
                abuf[r, 0:16] = a0
                abuf[r, 16:32] = a1
                return 0
            lax.fori_loop(0, WB, accum, 0)

            pltpu.sync_copy(abuf, out.at[pl.ds(hoff, WB)])
            if not is_last:
                pltpu.sync_copy(gbuf.at[0].at[pl.ds(0, WB)],
                                ebuf.at[pl.ds(hoff, WB)])
        plsc.subcore_barrier()

    layer(emb2, emb2, is_last=False)
    layer(ebuf, out, is_last=True)


def kernel(embedding, edge_weight, edge_index):
    row = edge_index[0].astype(jnp.int32)
    col = edge_index[1].astype(jnp.int32)
    w = edge_weight.astype(jnp.float32)

    emb2 = jnp.stack([embedding[:, :H], embedding[:, H:]], axis=0)
    emb2 = emb2.reshape(NC * N_NODES, H)

    colp = jnp.zeros((EP,), jnp.int32).at[:N_EDGES].set(col)
    rowp = jnp.full((EP,), N_NODES, jnp.int32).at[:N_EDGES].set(row).reshape(EP // CH, CH)
    wp = jnp.zeros((EP,), jnp.float32).at[:N_EDGES].set(w)

    out, _ = _lightgcn_sc(emb2, colp, rowp, wp)
    halves = out.reshape(NC, N_NODES, H)
    e_final = jnp.concatenate([halves[0], halves[1]], axis=1)
    return (e_final[:N_USERS], e_final[N_USERS:])
